# baseline (speedup 1.0000x reference)
import functools

import jax
import jax.numpy as jnp
from jax import lax
from jax.experimental import pallas as pl
from jax.experimental.pallas import tpu as pltpu
from jax.experimental.pallas import tpu_sc as plsc
from jax._src.pallas import mpmd as _mpmd

NNODES = 1000000
EMB_DIM = 64
B = 16384
L = 20

NC = 2
NS = 16
NW = NC * NS
SPW = B // NW
CHUNK_S = 32
CHUNK_W = CHUNK_S * L
NCHUNK = SPW // CHUNK_S
IDXW = 128

_mesh = plsc.VectorSubcoreMesh(core_axis_name="c", subcore_axis_name="s")
_sc_params = pltpu.CompilerParams(use_tc_tiling_on_sc=False, needs_layout_passes=False)


def _worker_id():
    return lax.axis_index("s") * NC + lax.axis_index("c")


_WROWS = NCHUNK * (CHUNK_W // IDXW)


@functools.partial(
    pl.kernel,
    out_type=(
        jax.ShapeDtypeStruct((B, EMB_DIM), jnp.float32),
        jax.ShapeDtypeStruct((B, EMB_DIM), jnp.float32),
    ),
    mesh=_mesh,
    scratch_types=[
        pltpu.VMEM((8, IDXW), jnp.int32),
        pltpu.VMEM((_WROWS, IDXW), jnp.int32),
        pltpu.VMEM((CHUNK_W, EMB_DIM), jnp.float32),
        pltpu.VMEM((CHUNK_W, EMB_DIM), jnp.float32),
        pltpu.VMEM((CHUNK_S, EMB_DIM), jnp.float32),
        pltpu.SemaphoreType.DMA,
        pltpu.SemaphoreType.DMA,
    ],
    compiler_params=_sc_params,
)
def _gather_agg(emb, start2, walk2, old_out, agg_out,
                sidx_v, widx_v, buf0, buf1, agg_v, semA, semB):
    cid = lax.axis_index("c")
    sid = lax.axis_index("s")
    wid = sid * NC + cid
    base = wid * SPW
    nrow = CHUNK_W // IDXW

    pltpu.sync_copy(start2.at[pl.ds(sid * 8, 8)], sidx_v)
    pltpu.sync_copy(walk2.at[pl.ds(wid * _WROWS, _WROWS)], widx_v)

    def fire(c, buf, sem):
        for j in range(nrow):
            pltpu.async_copy(
                emb.at[widx_v.at[c * nrow + j]],
                buf.at[pl.ds(j * IDXW, IDXW)], sem,
            )

    def drain(buf, sem):
        pltpu.make_async_copy(emb.at[pl.ds(0, CHUNK_W)], buf, sem).wait()

    def reduce_chunk(c, buf):
        def samp_body(s, carry2):
            r0 = s * L
            for q in range(EMB_DIM // 16):
                col = pl.ds(q * 16, 16)
                acc = buf[r0, col]
                for l in range(1, L):
                    acc = acc + buf[r0 + l, col]
                agg_v[s, col] = acc * (1.0 / L)
            return carry2

        lax.fori_loop(0, CHUNK_S, samp_body, 0)
        pltpu.sync_copy(agg_v, agg_out.at[pl.ds(base + c * CHUNK_S, CHUNK_S)])

    for j in range(4):
        pltpu.async_copy(
            emb.at[sidx_v.at[cid * 4 + j]],
            buf0.at[pl.ds(j * IDXW, IDXW)], semA,
        )
    fire(0, buf1, semB)
    pltpu.make_async_copy(
        emb.at[pl.ds(0, 4 * IDXW)], buf0.at[pl.ds(0, 4 * IDXW)], semA
    ).wait()
    pltpu.sync_copy(buf0.at[pl.ds(0, SPW)], old_out.at[pl.ds(base, SPW)])
    fire(1, buf0, semA)

    def body2(k, carry):
        c0 = 2 * k
        drain(buf1, semB)
        reduce_chunk(c0, buf1)

        @pl.when(k < NCHUNK // 2 - 1)
        def _():
            fire(c0 + 2, buf1, semB)

        drain(buf0, semA)
        reduce_chunk(c0 + 1, buf0)

        @pl.when(k < NCHUNK // 2 - 1)
        def _():
            fire(c0 + 3, buf0, semA)

        return carry

    lax.fori_loop(0, NCHUNK // 2, body2, 0)


def _combine_body(old_ref, agg_ref, ws_ref, wn_ref, new_ref, dist_ref):
    old = old_ref[...]
    agg = agg_ref[...]
    h = jnp.dot(old, ws_ref[...], preferred_element_type=jnp.float32)
    h = h + jnp.dot(agg, wn_ref[...], preferred_element_type=jnp.float32)
    h = jnp.maximum(h, 0.0)
    norm = jnp.sqrt(jnp.sum(h * h, axis=1, keepdims=True))
    new = h / (norm + 1e-8)
    new_ref[...] = new
    d = jnp.sqrt(jnp.sum((new - old) ** 2, axis=1) + 1e-12)
    dist_ref[...] = d.reshape(dist_ref.shape)


_RB = 1024


def _combine(old, agg, w_self, w_neigh):
    return pl.pallas_call(
        _combine_body,
        grid=(B // _RB,),
        in_specs=[
            pl.BlockSpec((_RB, EMB_DIM), lambda i: (i, 0)),
            pl.BlockSpec((_RB, EMB_DIM), lambda i: (i, 0)),
            pl.BlockSpec((EMB_DIM, EMB_DIM), lambda i: (0, 0)),
            pl.BlockSpec((EMB_DIM, EMB_DIM), lambda i: (0, 0)),
        ],
        out_specs=[
            pl.BlockSpec((_RB, EMB_DIM), lambda i: (i, 0)),
            pl.BlockSpec((_RB // 128, 128), lambda i: (i, 0)),
        ],
        out_shape=[
            jax.ShapeDtypeStruct((B, EMB_DIM), jnp.float32),
            jax.ShapeDtypeStruct((B // 128, 128), jnp.float32),
        ],
    )(old, agg, w_self, w_neigh)


def _scatter_body(mem_in, start2, newf, mem_out, sidx_v, rows_v, eidx_v):
    del mem_in
    cid = lax.axis_index("c")
    sid = lax.axis_index("s")
    wid = sid * NC + cid
    base = wid * SPW
    pltpu.sync_copy(start2.at[pl.ds(sid * 8, 8)], sidx_v)
    pltpu.sync_copy(newf.at[pl.ds(base * EMB_DIM, SPW * EMB_DIM)], rows_v)

    def samp_body(v, carry):
        row = cid * 4 + v // 8
        col0 = (v % 8) * 16
        rvec = sidx_v[row, pl.ds(col0, 16)] * EMB_DIM
        posv = (v * 16 + lax.iota(jnp.int32, 16)) * EMB_DIM
        for j in range(EMB_DIM):
            plsc.store_scatter(eidx_v, [posv + j], rvec + j)
        return carry

    lax.fori_loop(0, SPW // 16, samp_body, 0)
    pltpu.sync_copy(rows_v, mem_out.at[eidx_v])


_scatter = _mpmd._mpmd_map(
    [(_mesh, _scatter_body)],
    out_types=[jax.ShapeDtypeStruct((NNODES * EMB_DIM,), jnp.float32)],
    input_output_aliases={0: 0},
    scratch_types=[
        pltpu.VMEM((8, IDXW), jnp.int32),
        pltpu.VMEM((SPW * EMB_DIM,), jnp.float32),
        pltpu.VMEM((SPW * EMB_DIM,), jnp.int32),
    ],
    compiler_params=_sc_params,
)


def kernel(emb_features, W_self, W_neigh, start_idx, walk_idx):
    start2 = start_idx.reshape(B // IDXW, IDXW)
    walk2 = walk_idx.reshape(B * L // IDXW, IDXW)
    old_embs, agg = _gather_agg(emb_features, start2, walk2)
    new_embs, dist2 = _combine(old_embs, agg, W_self, W_neigh)
    emb_flat = emb_features.reshape(NNODES * EMB_DIM)
    new_flat = new_embs.reshape(B * EMB_DIM)
    (mem_flat,) = _scatter(emb_flat, start2, new_flat)
    mem_updated = mem_flat.reshape(NNODES, EMB_DIM)
    distances = dist2.reshape(B)
    return (new_embs, old_embs, mem_updated, distances)


# speedup vs baseline: 2.7260x; 2.7260x over previous
import functools

import jax
import jax.numpy as jnp
from jax import lax
from jax.experimental import pallas as pl
from jax.experimental.pallas import tpu as pltpu
from jax.experimental.pallas import tpu_sc as plsc
from jax._src.pallas import mpmd as _mpmd

NNODES = 1000000
EMB_DIM = 64
B = 16384
L = 20

NC = 2
NS = 16
NW = NC * NS
SPW = B // NW
CHUNK_S = 32
CHUNK_W = CHUNK_S * L
NCHUNK = SPW // CHUNK_S
IDXW = 128

_mesh = plsc.VectorSubcoreMesh(core_axis_name="c", subcore_axis_name="s")
_sc_params = pltpu.CompilerParams(use_tc_tiling_on_sc=False)


def _worker_id():
    return lax.axis_index("s") * NC + lax.axis_index("c")


_WROWS = NCHUNK * (CHUNK_W // IDXW)


@functools.partial(
    pl.kernel,
    out_type=(
        jax.ShapeDtypeStruct((B, EMB_DIM), jnp.float32),
        jax.ShapeDtypeStruct((B, EMB_DIM), jnp.float32),
    ),
    mesh=_mesh,
    scratch_types=[
        pltpu.VMEM((8, IDXW), jnp.int32),
        pltpu.VMEM((_WROWS, IDXW), jnp.int32),
        pltpu.VMEM((CHUNK_W, EMB_DIM), jnp.float32),
        pltpu.VMEM((CHUNK_W, EMB_DIM), jnp.float32),
        pltpu.VMEM((CHUNK_S, EMB_DIM), jnp.float32),
        pltpu.SemaphoreType.DMA,
        pltpu.SemaphoreType.DMA,
    ],
    compiler_params=_sc_params,
)
def _gather_agg(emb, start2, walk2, old_out, agg_out,
                sidx_v, widx_v, buf0, buf1, agg_v, semA, semB):
    cid = lax.axis_index("c")
    sid = lax.axis_index("s")
    wid = sid * NC + cid
    base = wid * SPW
    nrow = CHUNK_W // IDXW

    pltpu.sync_copy(start2.at[pl.ds(sid * 8, 8)], sidx_v)
    pltpu.sync_copy(walk2.at[pl.ds(wid * _WROWS, _WROWS)], widx_v)

    def fire(c, buf, sem):
        for j in range(nrow):
            pltpu.async_copy(
                emb.at[widx_v.at[c * nrow + j]],
                buf.at[pl.ds(j * IDXW, IDXW)], sem,
            )

    def drain(buf, sem):
        pltpu.make_async_copy(emb.at[pl.ds(0, CHUNK_W)], buf, sem).wait()

    def reduce_chunk(c, buf):
        def samp_body(s, carry2):
            r0 = s * L
            for q in range(EMB_DIM // 16):
                col = pl.ds(q * 16, 16)
                acc = buf[r0, col]
                for l in range(1, L):
                    acc = acc + buf[r0 + l, col]
                agg_v[s, col] = acc * (1.0 / L)
            return carry2

        lax.fori_loop(0, CHUNK_S, samp_body, 0)
        pltpu.sync_copy(agg_v, agg_out.at[pl.ds(base + c * CHUNK_S, CHUNK_S)])

    for j in range(4):
        pltpu.async_copy(
            emb.at[sidx_v.at[cid * 4 + j]],
            buf0.at[pl.ds(j * IDXW, IDXW)], semA,
        )
    fire(0, buf1, semB)
    pltpu.make_async_copy(
        emb.at[pl.ds(0, 4 * IDXW)], buf0.at[pl.ds(0, 4 * IDXW)], semA
    ).wait()
    pltpu.sync_copy(buf0.at[pl.ds(0, SPW)], old_out.at[pl.ds(base, SPW)])
    fire(1, buf0, semA)

    def body2(k, carry):
        c0 = 2 * k
        drain(buf1, semB)
        reduce_chunk(c0, buf1)

        @pl.when(k < NCHUNK // 2 - 1)
        def _():
            fire(c0 + 2, buf1, semB)

        drain(buf0, semA)
        reduce_chunk(c0 + 1, buf0)

        @pl.when(k < NCHUNK // 2 - 1)
        def _():
            fire(c0 + 3, buf0, semA)

        return carry

    lax.fori_loop(0, NCHUNK // 2, body2, 0)


def _combine_body(old_ref, agg_ref, ws_ref, wn_ref, new_ref, dist_ref):
    old = old_ref[...]
    agg = agg_ref[...]
    h = jnp.dot(old, ws_ref[...], preferred_element_type=jnp.float32)
    h = h + jnp.dot(agg, wn_ref[...], preferred_element_type=jnp.float32)
    h = jnp.maximum(h, 0.0)
    norm = jnp.sqrt(jnp.sum(h * h, axis=1, keepdims=True))
    new = h / (norm + 1e-8)
    new_ref[...] = new
    d = jnp.sqrt(jnp.sum((new - old) ** 2, axis=1) + 1e-12)
    dist_ref[...] = d.reshape(dist_ref.shape)


_RB = 1024


def _combine(old, agg, w_self, w_neigh):
    return pl.pallas_call(
        _combine_body,
        grid=(B // _RB,),
        in_specs=[
            pl.BlockSpec((_RB, EMB_DIM), lambda i: (i, 0)),
            pl.BlockSpec((_RB, EMB_DIM), lambda i: (i, 0)),
            pl.BlockSpec((EMB_DIM, EMB_DIM), lambda i: (0, 0)),
            pl.BlockSpec((EMB_DIM, EMB_DIM), lambda i: (0, 0)),
        ],
        out_specs=[
            pl.BlockSpec((_RB, EMB_DIM), lambda i: (i, 0)),
            pl.BlockSpec((_RB // 128, 128), lambda i: (i, 0)),
        ],
        out_shape=[
            jax.ShapeDtypeStruct((B, EMB_DIM), jnp.float32),
            jax.ShapeDtypeStruct((B // 128, 128), jnp.float32),
        ],
    )(old, agg, w_self, w_neigh)


def _scatter_body(mem_in, start2, new, mem_out, sidx_v, rows_v):
    del mem_in
    cid = lax.axis_index("c")
    sid = lax.axis_index("s")
    wid = sid * NC + cid
    base = wid * SPW
    pltpu.sync_copy(start2.at[pl.ds(sid * 8, 8)], sidx_v)
    pltpu.sync_copy(new.at[pl.ds(base, SPW)], rows_v)
    for j in range(4):
        pltpu.sync_copy(
            rows_v.at[pl.ds(j * IDXW, IDXW)], mem_out.at[sidx_v.at[cid * 4 + j]]
        )


_scatter = _mpmd._mpmd_map(
    [(_mesh, _scatter_body)],
    out_types=[jax.ShapeDtypeStruct((NNODES, EMB_DIM), jnp.float32)],
    input_output_aliases={0: 0},
    scratch_types=[
        pltpu.VMEM((8, IDXW), jnp.int32),
        pltpu.VMEM((SPW, EMB_DIM), jnp.float32),
    ],
    compiler_params=_sc_params,
)


def kernel(emb_features, W_self, W_neigh, start_idx, walk_idx):
    start2 = start_idx.reshape(B // IDXW, IDXW)
    walk2 = walk_idx.reshape(B * L // IDXW, IDXW)
    old_embs, agg = _gather_agg(emb_features, start2, walk2)
    new_embs, dist2 = _combine(old_embs, agg, W_self, W_neigh)
    (mem_updated,) = _scatter(emb_features, start2, new_embs)
    distances = dist2.reshape(B)
    return (new_embs, old_embs, mem_updated, distances)
